# Initial kernel scaffold; baseline (speedup 1.0000x reference)
import jax
import jax.numpy as jnp
from jax import lax
from jax.experimental import pallas as pl
from jax.experimental.pallas import tpu as pltpu
from jax.experimental.pallas import tpu_sc as plsc

N = 10000
NP = 10016
E_RAW = 320000
E_TOT = E_RAW + N
R2D = 2592
E_PAD = R2D * 128
RT = R2D // 16
D = 128
DH = 64
D2 = 16
G = 64


def _tc1_body(x_ref, w_ref, a_ref, h_ref, aa_ref):
    h = jnp.dot(x_ref[...], w_ref[...], preferred_element_type=jnp.float32)
    h_ref[...] = h
    aa_ref[...] = jnp.dot(h, a_ref[...], preferred_element_type=jnp.float32)


def _tc1(x, W1, A1):
    return pl.pallas_call(
        _tc1_body,
        grid=(10,),
        in_specs=[
            pl.BlockSpec((1000, 128), lambda i: (i, 0)),
            pl.BlockSpec((128, 128), lambda i: (0, 0)),
            pl.BlockSpec((128, 128), lambda i: (0, 0)),
        ],
        out_specs=[
            pl.BlockSpec((1000, 128), lambda i: (i, 0)),
            pl.BlockSpec((1000, 128), lambda i: (i, 0)),
        ],
        out_shape=[
            jax.ShapeDtypeStruct((N, 128), jnp.float32),
            jax.ShapeDtypeStruct((N, 128), jnp.float32),
        ],
    )(x, W1, A1)


def _tc2_body(o_ref, b_ref, w_ref, a_ref, h_ref, aa_ref):
    hr = jnp.maximum(o_ref[...] + b_ref[...], 0.0)
    h2 = jnp.dot(hr, w_ref[...], preferred_element_type=jnp.float32)
    h_ref[...] = h2
    aa_ref[...] = jnp.dot(h2, a_ref[...], preferred_element_type=jnp.float32)


def _tc2(out1, b1, W2p, A2p):
    return pl.pallas_call(
        _tc2_body,
        grid=(10,),
        in_specs=[
            pl.BlockSpec((1000, 128), lambda i: (i, 0)),
            pl.BlockSpec((1, 128), lambda i: (0, 0)),
            pl.BlockSpec((128, 128), lambda i: (0, 0)),
            pl.BlockSpec((128, 128), lambda i: (0, 0)),
        ],
        out_specs=[
            pl.BlockSpec((1000, 128), lambda i: (i, 0)),
            pl.BlockSpec((1000, 128), lambda i: (i, 0)),
        ],
        out_shape=[
            jax.ShapeDtypeStruct((N, 128), jnp.float32),
            jax.ShapeDtypeStruct((N, 128), jnp.float32),
        ],
    )(out1, b1, W2p, A2p)


def _tc3_body(p0_ref, p1_ref, bat_ref, b2_ref, out_ref):
    hsum = p0_ref[...] + p1_ref[...]
    bat = bat_ref[...]
    gid = lax.broadcasted_iota(jnp.int32, (N, G), 1)
    oh = (bat == gid).astype(jnp.float32)
    sums = lax.dot_general(oh, hsum, (((0,), (0,)), ((), ())),
                           preferred_element_type=jnp.float32)
    cnt = jnp.sum(oh, axis=0)
    mean = sums / jnp.maximum(cnt, 1.0)[:, None] + b2_ref[...]
    m = jnp.max(mean, axis=-1, keepdims=True)
    z = mean - m
    lse = jnp.log(jnp.sum(jnp.exp(z), axis=-1, keepdims=True))
    out_ref[...] = z - lse


def _tc3(p0, p1, bat2d, b2r):
    return pl.pallas_call(
        _tc3_body,
        out_shape=jax.ShapeDtypeStruct((G, D2), jnp.float32),
    )(p0, p1, bat2d, b2r)


_MESH = dict(core_axis_name="c", subcore_axis_name="s")


def _edge_scalar_phase(src_l, dst_l, as_l, ad_l, den_l, w_l, n_rows):
    def row_body(r, carry):
        for k in range(8):
            sv = src_l[r, pl.ds(k * 16, 16)]
            dv = dst_l[r, pl.ds(k * 16, 16)]
            a = plsc.load_gather(as_l, [sv]) + plsc.load_gather(ad_l, [dv])
            a = jnp.maximum(a, 0.2 * a)
            w = jnp.exp(a)
            w_l[r, pl.ds(k * 16, 16)] = w
            plsc.addupdate_scatter(den_l, [dv], w)
        return carry
    lax.fori_loop(0, n_rows, row_body, 0)


def _l1_body(h1p, as_h, ad_h, src_h, dst_h, znd, zn, out_h,
             h_sh, out_sh, den_sh, as_l, ad_l, den_l,
             src_l, dst_l, w_l, cbuf, rows, sem_g, sem_s):
    cid = lax.axis_index("c")
    sid = lax.axis_index("s")

    @pl.when(sid == 0)
    def _():
        pltpu.sync_copy(h1p.at[:, pl.ds(cid * DH, DH)], h_sh)

    @pl.when(sid == 1)
    def _():
        pltpu.sync_copy(znd, out_sh)

    @pl.when(sid == 2)
    def _():
        pltpu.sync_copy(zn, den_sh)

    pltpu.sync_copy(as_h, as_l)
    pltpu.sync_copy(ad_h, ad_l)
    pltpu.sync_copy(zn, den_l)
    base = sid * RT
    pltpu.sync_copy(src_h.at[pl.ds(base, RT)], src_l)
    pltpu.sync_copy(dst_h.at[pl.ds(base, RT)], dst_l)

    _edge_scalar_phase(src_l, dst_l, as_l, ad_l, den_l, w_l, RT)

    plsc.subcore_barrier()
    pltpu.sync_copy(den_l, den_sh, add=True)
    plsc.subcore_barrier()
    pltpu.sync_copy(den_sh, den_l)

    def batch_body(j, carry):
        for k in range(8):
            dv = dst_l[j, pl.ds(k * 16, 16)]
            d = plsc.load_gather(den_l, [dv])
            w = w_l[j, pl.ds(k * 16, 16)]
            cbuf[pl.ds(k * 16, 16)] = w / (d + 1e-16)
        pltpu.async_copy(h_sh.at[src_l.at[j]], rows, sem_g).wait()

        def scale_body(r, c2):
            c = cbuf[r]
            for k in range(4):
                rows[r, pl.ds(k * 16, 16)] = rows[r, pl.ds(k * 16, 16)] * c
            return c2
        lax.fori_loop(0, 128, scale_body, 0)
        pltpu.async_copy(rows, out_sh.at[dst_l.at[j]], sem_s, add=True).wait()
        return carry
    lax.fori_loop(0, RT, batch_body, 0)

    plsc.subcore_barrier()
    pltpu.sync_copy(out_sh.at[pl.ds(sid * 625, 625)],
                    out_h.at[pl.ds(sid * 625, 625), pl.ds(cid * DH, DH)])


def _l1(h1p, as1p, ad1p, src2d, dst2d, znd, zn):
    return pl.kernel(
        _l1_body,
        out_type=jax.ShapeDtypeStruct((N, D), jnp.float32),
        mesh=plsc.VectorSubcoreMesh(**_MESH),
        scratch_types=[
            pltpu.VMEM_SHARED((NP, DH), jnp.float32),
            pltpu.VMEM_SHARED((NP, DH), jnp.float32),
            pltpu.VMEM_SHARED((NP,), jnp.float32),
            pltpu.VMEM((NP,), jnp.float32),
            pltpu.VMEM((NP,), jnp.float32),
            pltpu.VMEM((NP,), jnp.float32),
            pltpu.VMEM((RT, 128), jnp.int32),
            pltpu.VMEM((RT, 128), jnp.int32),
            pltpu.VMEM((RT, 128), jnp.float32),
            pltpu.VMEM((128,), jnp.float32),
            pltpu.VMEM((128, DH), jnp.float32),
            pltpu.SemaphoreType.DMA,
            pltpu.SemaphoreType.DMA,
        ],
    )(h1p, as1p, ad1p, src2d, dst2d, znd, zn)


def _l2_body(h2p, as_h, ad_h, src_h, dst_h, znd, zn, p_h,
             h_sh, out_sh, den_sh, as_l, ad_l, den_l,
             src_l, dst_l, w_l, cbuf, rows, sem_g, sem_s):
    cid = lax.axis_index("c")
    sid = lax.axis_index("s")

    @pl.when(sid == 0)
    def _():
        pltpu.sync_copy(h2p, h_sh)

    @pl.when(sid == 1)
    def _():
        pltpu.sync_copy(znd, out_sh)

    @pl.when(sid == 2)
    def _():
        pltpu.sync_copy(zn, den_sh)

    pltpu.sync_copy(as_h, as_l)
    pltpu.sync_copy(ad_h, ad_l)
    pltpu.sync_copy(zn, den_l)
    base = sid * RT
    pltpu.sync_copy(src_h.at[pl.ds(base, RT)], src_l)
    pltpu.sync_copy(dst_h.at[pl.ds(base, RT)], dst_l)

    _edge_scalar_phase(src_l, dst_l, as_l, ad_l, den_l, w_l, RT)

    plsc.subcore_barrier()
    pltpu.sync_copy(den_l, den_sh, add=True)
    plsc.subcore_barrier()
    pltpu.sync_copy(den_sh, den_l)

    half = RT // 2

    def batch_body(j, carry):
        r = cid * half + j
        for k in range(8):
            dv = dst_l[r, pl.ds(k * 16, 16)]
            d = plsc.load_gather(den_l, [dv])
            w = w_l[r, pl.ds(k * 16, 16)]
            cbuf[pl.ds(k * 16, 16)] = w / (d + 1e-16)
        pltpu.async_copy(h_sh.at[src_l.at[r]], rows, sem_g).wait()

        def scale_body(rr, c2):
            rows[rr] = rows[rr] * cbuf[rr]
            return c2
        lax.fori_loop(0, 128, scale_body, 0)
        pltpu.async_copy(rows, out_sh.at[dst_l.at[r]], sem_s, add=True).wait()
        return carry
    lax.fori_loop(0, half, batch_body, 0)

    plsc.subcore_barrier()
    pltpu.sync_copy(out_sh.at[pl.ds(sid * 625, 625)],
                    p_h.at[cid, pl.ds(sid * 625, 625)])


def _l2(h2p, as2p, ad2p, src2d, dst2d, znd2, zn):
    return pl.kernel(
        _l2_body,
        out_type=jax.ShapeDtypeStruct((2, N, D2), jnp.float32),
        mesh=plsc.VectorSubcoreMesh(**_MESH),
        scratch_types=[
            pltpu.VMEM_SHARED((NP, D2), jnp.float32),
            pltpu.VMEM_SHARED((NP, D2), jnp.float32),
            pltpu.VMEM_SHARED((NP,), jnp.float32),
            pltpu.VMEM((NP,), jnp.float32),
            pltpu.VMEM((NP,), jnp.float32),
            pltpu.VMEM((NP,), jnp.float32),
            pltpu.VMEM((RT, 128), jnp.int32),
            pltpu.VMEM((RT, 128), jnp.int32),
            pltpu.VMEM((RT, 128), jnp.float32),
            pltpu.VMEM((128,), jnp.float32),
            pltpu.VMEM((128, D2), jnp.float32),
            pltpu.SemaphoreType.DMA,
            pltpu.SemaphoreType.DMA,
        ],
    )(h2p, as2p, ad2p, src2d, dst2d, znd2, zn)


def _pad_nodes(v):
    return jnp.pad(v, (0, NP - N))


@jax.jit
def kernel(x, edge_index, batch, W1, a_src1, a_dst1, b1, W2, a_src2, a_dst2, b2):
    ei = edge_index.astype(jnp.int32)
    loop = jnp.arange(N, dtype=jnp.int32)
    pad = jnp.full((E_PAD - E_TOT,), N, dtype=jnp.int32)
    src2d = jnp.concatenate([ei[0], loop, pad]).reshape(R2D, 128)
    dst2d = jnp.concatenate([ei[1], loop, pad]).reshape(R2D, 128)

    A1 = jnp.zeros((128, 128), jnp.float32).at[:, 0].set(a_src1).at[:, 1].set(a_dst1)
    W2p = jnp.zeros((128, 128), jnp.float32).at[:, :D2].set(W2)
    A2p = jnp.zeros((128, 128), jnp.float32).at[:D2, 0].set(a_src2).at[:D2, 1].set(a_dst2)

    znd = jnp.zeros((NP, DH), jnp.float32)
    znd2 = jnp.zeros((NP, D2), jnp.float32)
    zn = jnp.zeros((NP,), jnp.float32)

    h1, aa1 = _tc1(x, W1, A1)
    h1p = jnp.pad(h1, ((0, NP - N), (0, 0)))
    out1 = _l1(h1p, _pad_nodes(aa1[:, 0]), _pad_nodes(aa1[:, 1]),
               src2d, dst2d, znd, zn)

    h2f, aa2 = _tc2(out1, b1.reshape(1, 128), W2p, A2p)
    h2p = jnp.pad(h2f[:, :D2], ((0, NP - N), (0, 0)))
    p = _l2(h2p, _pad_nodes(aa2[:, 0]), _pad_nodes(aa2[:, 1]),
            src2d, dst2d, znd2, zn)

    return _tc3(p[0], p[1], batch.astype(jnp.int32).reshape(N, 1),
                b2.reshape(1, D2))


# speedup vs baseline: 7.6256x; 7.6256x over previous
import jax
import jax.numpy as jnp
from jax import lax
from jax.experimental import pallas as pl
from jax.experimental.pallas import tpu as pltpu
from jax.experimental.pallas import tpu_sc as plsc

N = 10000
NP = 10016
E_RAW = 320000
E_TOT = E_RAW + N
RPT = 176
E_PAD = RPT * 16 * 128
CH = 8
D = 128
D2 = 16
G = 64


def _tc1_body(x_ref, w_ref, a_ref, h_ref, aa_ref):
    h = jnp.dot(x_ref[...], w_ref[...], preferred_element_type=jnp.float32)
    h_ref[...] = h
    aa_ref[...] = jnp.dot(h, a_ref[...], preferred_element_type=jnp.float32)


def _tc1(x, W1, A1):
    return pl.pallas_call(
        _tc1_body,
        grid=(10,),
        in_specs=[
            pl.BlockSpec((1000, 128), lambda i: (i, 0)),
            pl.BlockSpec((128, 128), lambda i: (0, 0)),
            pl.BlockSpec((128, 128), lambda i: (0, 0)),
        ],
        out_specs=[
            pl.BlockSpec((1000, 128), lambda i: (i, 0)),
            pl.BlockSpec((1000, 128), lambda i: (i, 0)),
        ],
        out_shape=[
            jax.ShapeDtypeStruct((N, 128), jnp.float32),
            jax.ShapeDtypeStruct((N, 128), jnp.float32),
        ],
    )(x, W1, A1)


def _tc2_body(p0_ref, p1_ref, b_ref, w_ref, a_ref, h_ref, aa_ref):
    hr = jnp.maximum(p0_ref[...] + p1_ref[...] + b_ref[...], 0.0)
    h2 = jnp.dot(hr, w_ref[...], preferred_element_type=jnp.float32)
    h_ref[...] = h2
    aa_ref[...] = jnp.dot(h2, a_ref[...], preferred_element_type=jnp.float32)


def _tc2(p0, p1, b1, W2p, A2p):
    return pl.pallas_call(
        _tc2_body,
        grid=(10,),
        in_specs=[
            pl.BlockSpec((1000, 128), lambda i: (i, 0)),
            pl.BlockSpec((1000, 128), lambda i: (i, 0)),
            pl.BlockSpec((1, 128), lambda i: (0, 0)),
            pl.BlockSpec((128, 128), lambda i: (0, 0)),
            pl.BlockSpec((128, 128), lambda i: (0, 0)),
        ],
        out_specs=[
            pl.BlockSpec((1000, 128), lambda i: (i, 0)),
            pl.BlockSpec((1000, 128), lambda i: (i, 0)),
        ],
        out_shape=[
            jax.ShapeDtypeStruct((N, 128), jnp.float32),
            jax.ShapeDtypeStruct((N, 128), jnp.float32),
        ],
    )(p0, p1, b1, W2p, A2p)


def _tc3_body(p0_ref, p1_ref, bat_ref, b2_ref, out_ref):
    hsum = p0_ref[...] + p1_ref[...]
    bat = bat_ref[...]
    gid = lax.broadcasted_iota(jnp.int32, (N, G), 1)
    oh = (bat == gid).astype(jnp.float32)
    sums = lax.dot_general(oh, hsum, (((0,), (0,)), ((), ())),
                           preferred_element_type=jnp.float32)
    cnt = jnp.sum(oh, axis=0)
    mean = sums / jnp.maximum(cnt, 1.0)[:, None] + b2_ref[...]
    m = jnp.max(mean, axis=-1, keepdims=True)
    z = mean - m
    lse = jnp.log(jnp.sum(jnp.exp(z), axis=-1, keepdims=True))
    out_ref[...] = z - lse


def _tc3(p0, p1, bat2d, b2r):
    return pl.pallas_call(
        _tc3_body,
        out_shape=jax.ShapeDtypeStruct((G, D2), jnp.float32),
    )(p0, p1, bat2d, b2r)


def _gat_body(hp, asp, adp, src_h, dst_h, zn, znd, out_h,
              out_sh, den_sh, as_l, ad_l, src_c, dst_c,
              wbuf, deng, cbuf, rows, sem_g, sem_s):
    cid = lax.axis_index("c")
    sid = lax.axis_index("s")

    @pl.when(sid == 0)
    def _():
        pltpu.sync_copy(znd, out_sh)

    @pl.when(sid == 1)
    def _():
        pltpu.sync_copy(zn, den_sh)

    pltpu.sync_copy(asp, as_l)
    pltpu.sync_copy(adp, ad_l)
    plsc.subcore_barrier()

    def sc_chunk(ch, carry):
        pltpu.sync_copy(src_h.at[sid, pl.ds(ch * CH, CH)], src_c)
        pltpu.sync_copy(dst_h.at[sid, pl.ds(ch * CH, CH)], dst_c)

        def sc_row(r, c2):
            for k in range(8):
                sv = src_c[r, pl.ds(k * 16, 16)]
                dv = dst_c[r, pl.ds(k * 16, 16)]
                a = plsc.load_gather(as_l, [sv]) + plsc.load_gather(ad_l, [dv])
                a = jnp.maximum(a, 0.2 * a)
                wbuf[pl.ds(k * 16, 16)] = jnp.exp(a)
            pltpu.sync_copy(wbuf, den_sh.at[dst_c.at[r]], add=True)
            return c2
        lax.fori_loop(0, CH, sc_row, 0)
        return carry
    lax.fori_loop(0, RPT // CH, sc_chunk, 0)

    plsc.subcore_barrier()

    half = RPT // 2
    base = cid * half

    def row_chunk(ch, carry):
        pltpu.sync_copy(src_h.at[sid, pl.ds(base + ch * CH, CH)], src_c)
        pltpu.sync_copy(dst_h.at[sid, pl.ds(base + ch * CH, CH)], dst_c)

        def row_body(r, c2):
            pltpu.sync_copy(den_sh.at[dst_c.at[r]], deng)
            for k in range(8):
                sv = src_c[r, pl.ds(k * 16, 16)]
                dv = dst_c[r, pl.ds(k * 16, 16)]
                a = plsc.load_gather(as_l, [sv]) + plsc.load_gather(ad_l, [dv])
                a = jnp.maximum(a, 0.2 * a)
                w = jnp.exp(a)
                cbuf[pl.ds(k * 16, 16)] = w / (deng[pl.ds(k * 16, 16)] + 1e-16)
            pltpu.async_copy(hp.at[src_c.at[r]], rows, sem_g).wait()

            def scale_body(q, c3):
                c = cbuf[pl.ds(q, 16)][0]
                for k in range(8):
                    rows[q, pl.ds(k * 16, 16)] = rows[q, pl.ds(k * 16, 16)] * c
                return c3
            lax.fori_loop(0, 128, scale_body, 0)
            pltpu.async_copy(rows, out_sh.at[dst_c.at[r]], sem_s, add=True).wait()
            return c2
        lax.fori_loop(0, CH, row_body, 0)
        return carry
    lax.fori_loop(0, half // CH, row_chunk, 0)

    plsc.subcore_barrier()

    @pl.when(sid == 0)
    def _():
        pltpu.sync_copy(out_sh, out_h.at[cid])


def _gat_sc(hp, asp, adp, src3d, dst3d, zn, znd):
    return pl.kernel(
        _gat_body,
        out_type=jax.ShapeDtypeStruct((2, NP, D), jnp.float32),
        mesh=plsc.VectorSubcoreMesh(core_axis_name="c", subcore_axis_name="s"),
        compiler_params=pltpu.CompilerParams(needs_layout_passes=False),
        scratch_types=[
            pltpu.VMEM_SHARED((NP, D), jnp.float32),
            pltpu.VMEM_SHARED((NP,), jnp.float32),
            pltpu.VMEM((NP,), jnp.float32),
            pltpu.VMEM((NP,), jnp.float32),
            pltpu.VMEM((CH, 128), jnp.int32),
            pltpu.VMEM((CH, 128), jnp.int32),
            pltpu.VMEM((128,), jnp.float32),
            pltpu.VMEM((128,), jnp.float32),
            pltpu.VMEM((144,), jnp.float32),
            pltpu.VMEM((128, D), jnp.float32),
            pltpu.SemaphoreType.DMA,
            pltpu.SemaphoreType.DMA,
        ],
    )(hp, asp, adp, src3d, dst3d, zn, znd)


def _pad_nodes(v):
    return jnp.pad(v, (0, NP - N))


@jax.jit
def kernel(x, edge_index, batch, W1, a_src1, a_dst1, b1, W2, a_src2, a_dst2, b2):
    ei = edge_index.astype(jnp.int32)
    loop = jnp.arange(N, dtype=jnp.int32)
    pad = jnp.full((E_PAD - E_TOT,), N, dtype=jnp.int32)
    src3d = jnp.concatenate([ei[0], loop, pad]).reshape(16, RPT, 128)
    dst3d = jnp.concatenate([ei[1], loop, pad]).reshape(16, RPT, 128)

    A1 = jnp.zeros((128, 128), jnp.float32).at[:, 0].set(a_src1).at[:, 1].set(a_dst1)
    W2p = jnp.zeros((128, 128), jnp.float32).at[:, :D2].set(W2)
    A2p = jnp.zeros((128, 128), jnp.float32).at[:D2, 0].set(a_src2).at[:D2, 1].set(a_dst2)

    zn = jnp.zeros((NP,), jnp.float32)
    znd = jnp.zeros((NP, D), jnp.float32)

    h1, aa1 = _tc1(x, W1, A1)
    h1p = jnp.pad(h1, ((0, NP - N), (0, 0)))
    o1 = _gat_sc(h1p, _pad_nodes(aa1[:, 0]), _pad_nodes(aa1[:, 1]),
                 src3d, dst3d, zn, znd)

    h2f, aa2 = _tc2(o1[0, :N], o1[1, :N], b1.reshape(1, 128), W2p, A2p)
    h2p = jnp.pad(h2f, ((0, NP - N), (0, 0)))
    o2 = _gat_sc(h2p, _pad_nodes(aa2[:, 0]), _pad_nodes(aa2[:, 1]),
                 src3d, dst3d, zn, znd)

    return _tc3(o2[0, :N, :D2], o2[1, :N, :D2],
                batch.astype(jnp.int32).reshape(N, 1), b2.reshape(1, D2))
